# baseline (speedup 1.0000x reference)
import jax
import jax.numpy as jnp
from jax import lax
from jax.experimental import pallas as pl
from jax.experimental.pallas import tpu as pltpu
from jax.experimental.pallas import tpu_sc as plsc

W1 = 1e-07
W2 = 1.0
W3 = 1e-07
W4 = 1.0
NEG_WEIGHT = 200.0
GAMMA = 1e-4
LAMBDA = 1e-3

_NC = 2
_NS = 16
_NW = _NC * _NS
_L = 16
_DP = 128


def _softplus(x):
    return jnp.maximum(x, 0.0) + jnp.log1p(jnp.exp(-jnp.abs(x)))


def _make_sc_kernel(B, NEG, D, K, I_NUM):
    assert B % _NW == 0
    BPW = B // _NW
    NJG = (NEG + _L - 1) // _L
    NEG_PAD = NJG * _L

    mesh = plsc.VectorSubcoreMesh(
        core_axis_name="c", subcore_axis_name="s",
        num_cores=_NC, num_subcores=_NS)

    out_type = (
        jax.ShapeDtypeStruct((B, _L), jnp.float32),
        jax.ShapeDtypeStruct((B, _L), jnp.float32),
        jax.ShapeDtypeStruct((B * NEG,), jnp.float32),
        jax.ShapeDtypeStruct((B * NEG,), jnp.float32),
        jax.ShapeDtypeStruct((B,), jnp.float32),
        jax.ShapeDtypeStruct((B,), jnp.float32),
    )
    scratch = [
        pltpu.VMEM((BPW,), jnp.int32),
        pltpu.VMEM((BPW,), jnp.int32),
        pltpu.VMEM((BPW,), jnp.int32),
        pltpu.VMEM((BPW, _DP), jnp.float32),
        pltpu.VMEM((K, BPW), jnp.int32),
        pltpu.VMEM((K, BPW), jnp.float32),
        pltpu.VMEM((BPW,), jnp.float32),
        pltpu.VMEM((BPW,), jnp.float32),
        pltpu.VMEM((NEG,), jnp.int32),
        pltpu.VMEM((NEG, _DP), jnp.float32),
        pltpu.VMEM((NEG_PAD,), jnp.float32),
        pltpu.VMEM((NEG,), jnp.float32),
        pltpu.VMEM((_L,), jnp.int32),
        pltpu.VMEM((_L, _DP), jnp.float32),
        pltpu.VMEM((BPW, _L), jnp.float32),
        pltpu.VMEM((BPW, _L), jnp.float32),
        pltpu.SemaphoreType.DMA,
        pltpu.SemaphoreType.DMA,
    ]

    def body(users_h, pos_h, neg_h, uep_h, iep_h, bu_h, bi_h, iicf_h, iinf_h,
             posinner_o, simpad_o, negsc_o, betain_o, betau_o, betaip_o,
             users_v, pos_v, idxtmp_v, urows_v, nbmat_v, simmat_v,
             betau_v, betaip_v, negidx_v, negrows_v, negsc_v, betain_v,
             combidx_v, pirows_v, posinner_v, simpad_v, sem, sem2):
        wid = lax.axis_index("c") * _NS + lax.axis_index("s")
        base = wid * BPW
        lane = lax.iota(jnp.int32, _L)

        pltpu.sync_copy(users_h.at[pl.ds(base, BPW)], users_v)
        pltpu.sync_copy(pos_h.at[pl.ds(base, BPW)], pos_v)
        c1 = pltpu.async_copy(uep_h.at[users_v], urows_v, sem)
        c4 = pltpu.async_copy(bu_h.at[users_v], betau_v, sem)
        c5 = pltpu.async_copy(bi_h.at[pos_v], betaip_v, sem)
        c1.wait(); c4.wait(); c5.wait()
        for k in range(K):
            for t in range(BPW // _L):
                sl = pl.ds(t * _L, _L)
                idxtmp_v[sl] = pos_v[sl] + jnp.full((_L,), k * I_NUM, jnp.int32)
            gn = pltpu.async_copy(iinf_h.at[idxtmp_v], nbmat_v.at[k], sem)
            gs = pltpu.async_copy(iicf_h.at[idxtmp_v], simmat_v.at[k], sem)
            gn.wait(); gs.wait()
        pltpu.sync_copy(betau_v, betau_o.at[pl.ds(base, BPW)])
        pltpu.sync_copy(betaip_v, betaip_o.at[pl.ds(base, BPW)])

        rowidx = [jnp.minimum(jg * _L + lane, NEG - 1) for jg in range(NJG)]
        nbcol = jnp.clip(lane - 1, 0, K - 1)
        nbmask = (lane >= 1) & (lane <= K)

        def b_step(bl, carry):
            bg = base + bl
            pltpu.sync_copy(neg_h.at[pl.ds(bg * NEG, NEG)], negidx_v)
            g1 = pltpu.async_copy(iep_h.at[negidx_v.at[pl.ds(0, 128)]],
                                  negrows_v.at[pl.ds(0, 128)], sem)
            g2 = pltpu.async_copy(iep_h.at[negidx_v.at[pl.ds(128, NEG - 128)]],
                                  negrows_v.at[pl.ds(128, NEG - 128)], sem)
            g3 = pltpu.async_copy(bi_h.at[negidx_v.at[pl.ds(0, 128)]],
                                  betain_v.at[pl.ds(0, 128)], sem)
            g4 = pltpu.async_copy(bi_h.at[negidx_v.at[pl.ds(128, NEG - 128)]],
                                  betain_v.at[pl.ds(128, NEG - 128)], sem)

            bl_vec = jnp.full((_L,), bl, jnp.int32)
            cand = plsc.load_gather(nbmat_v, [nbcol, bl_vec])
            posb = plsc.load_gather(pos_v, [bl_vec])
            comb = jnp.where(nbmask, cand, posb)
            combidx_v[...] = comb
            g5 = pltpu.async_copy(iep_h.at[combidx_v], pirows_v, sem2)

            simv = plsc.load_gather(simmat_v, [nbcol, bl_vec])
            simv = jnp.where(nbmask, simv, 0.0)
            simpad_v[bl] = simv

            g1.wait(); g2.wait(); g3.wait(); g4.wait(); g5.wait()

            accs = [jnp.zeros((_L,), jnp.float32) for _ in range(NJG)]
            pacc = jnp.zeros((_L,), jnp.float32)
            for t in range(D // _L):
                uvec = urows_v[bl, pl.ds(t * _L, _L)]
                for dd in range(_L):
                    d = t * _L + dd
                    u_d = uvec[dd]
                    dvec = jnp.full((_L,), d, jnp.int32)
                    for jg in range(NJG):
                        v = plsc.load_gather(negrows_v, [rowidx[jg], dvec])
                        accs[jg] = accs[jg] + v * u_d
                    pv = plsc.load_gather(pirows_v, [lane, dvec])
                    pacc = pacc + pv * u_d

            for jg in range(NJG):
                negsc_v[pl.ds(jg * _L, _L)] = accs[jg]
            posinner_v[bl] = pacc

            pltpu.sync_copy(negsc_v.at[pl.ds(0, NEG)],
                            negsc_o.at[pl.ds(bg * NEG, NEG)])
            pltpu.sync_copy(betain_v, betain_o.at[pl.ds(bg * NEG, NEG)])
            return carry

        lax.fori_loop(0, BPW, b_step, 0)

        pltpu.sync_copy(posinner_v, posinner_o.at[pl.ds(base, BPW)])
        pltpu.sync_copy(simpad_v, simpad_o.at[pl.ds(base, BPW)])

    return pl.kernel(body, out_type=out_type, mesh=mesh,
                     scratch_types=scratch,
                     compiler_params=pltpu.CompilerParams(
                         needs_layout_passes=False))


def _make_norm_body(ncols, block):
    def _norm_body(ue_r, ie_r, out_r):
        g = pl.program_id(0)

        @pl.when(g == 0)
        def _init():
            out_r[0, 0] = 0.0

        x = ue_r[...]
        y = ie_r[...]
        col = g * block + lax.broadcasted_iota(jnp.int32, x.shape, 1)
        valid = col < ncols
        out_r[0, 0] += (jnp.sum(jnp.where(valid, x * x, 0.0))
                        + jnp.sum(jnp.where(valid, y * y, 0.0)))
    return _norm_body


def _final_body(betau_r, betaip_r, posinner_r, simpad_r, negsc_r, betain_r,
                norm_r, out_r):
    bu = betau_r[...]
    bip = betaip_r[...]
    pin = posinner_r[...]
    simp = simpad_r[...]
    ns = negsc_r[...]
    bin_ = betain_r[...]
    NEG = ns.shape[1]

    neg_w = W3 + W4 * bu * bin_
    neg_term = (NEG_WEIGHT / NEG) * jnp.sum(neg_w * _softplus(ns))

    pow_w = W1 + W2 * bu * bip
    col = lax.broadcasted_iota(jnp.int32, pin.shape, 1)
    wmat = jnp.where(col == 0, pow_w, 0.0) + LAMBDA * simp
    pos_term = jnp.sum(wmat * _softplus(-pin))

    out_r[0, 0] = pos_term + neg_term + GAMMA * 0.5 * norm_r[0, 0]


def kernel(users, pos_items, neg_items, user_embeds, item_embeds,
           beta_uD, beta_iD, ii_constraint_mat, ii_neighbor_mat):
    B = users.shape[0]
    NEG = neg_items.shape[1]
    D = user_embeds.shape[1]
    K = ii_neighbor_mat.shape[1]
    U_NUM = user_embeds.shape[0]
    I_NUM = item_embeds.shape[0]

    users = users.astype(jnp.int32)
    pos_items = pos_items.astype(jnp.int32)
    neg_flat = neg_items.reshape(-1).astype(jnp.int32)

    ue_pad = jnp.pad(user_embeds, ((0, 0), (0, _DP - D)))
    ie_pad = jnp.pad(item_embeds, ((0, 0), (0, _DP - D)))
    iin_flat = ii_neighbor_mat.astype(jnp.int32).T.reshape(-1)
    iic_flat = ii_constraint_mat.T.reshape(-1)

    sc = _make_sc_kernel(B, NEG, D, K, I_NUM)
    posinner, simpad, negsc, betain, betau, betaip = sc(
        users, pos_items, neg_flat, ue_pad, ie_pad,
        beta_uD, beta_iD, iic_flat, iin_flat)

    assert U_NUM == I_NUM
    BLK = 128 * 61
    G = -(-U_NUM // BLK)
    norm = pl.pallas_call(
        _make_norm_body(U_NUM, BLK),
        grid=(G,),
        in_specs=[
            pl.BlockSpec((D, BLK), lambda g: (0, g)),
            pl.BlockSpec((D, BLK), lambda g: (0, g)),
        ],
        out_specs=pl.BlockSpec((1, 1), lambda g: (0, 0),
                               memory_space=pltpu.SMEM),
        out_shape=jax.ShapeDtypeStruct((1, 1), jnp.float32),
    )(user_embeds.T, item_embeds.T)

    full = lambda s: pl.BlockSpec(s, lambda: tuple(0 for _ in s))
    loss = pl.pallas_call(
        _final_body,
        in_specs=[
            full((B, 1)), full((B, 1)), full((B, _L)), full((B, _L)),
            full((B, NEG)), full((B, NEG)),
            pl.BlockSpec((1, 1), lambda: (0, 0), memory_space=pltpu.SMEM),
        ],
        out_specs=pl.BlockSpec((1, 1), lambda: (0, 0),
                               memory_space=pltpu.SMEM),
        out_shape=jax.ShapeDtypeStruct((1, 1), jnp.float32),
    )(betau.reshape(B, 1), betaip.reshape(B, 1), posinner, simpad,
      negsc.reshape(B, NEG), betain.reshape(B, NEG), norm)

    return loss[0, 0]


# speedup vs baseline: 1.0156x; 1.0156x over previous
import jax
import jax.numpy as jnp
from jax import lax
from jax.experimental import pallas as pl
from jax.experimental.pallas import tpu as pltpu
from jax.experimental.pallas import tpu_sc as plsc

W1 = 1e-07
W2 = 1.0
W3 = 1e-07
W4 = 1.0
NEG_WEIGHT = 200.0
GAMMA = 1e-4
LAMBDA = 1e-3

_NC = 2
_NS = 16
_NW = _NC * _NS
_L = 16
_DP = 128


def _softplus(x):
    return jnp.maximum(x, 0.0) + jnp.log1p(jnp.exp(-jnp.abs(x)))


def _make_sc_kernel(B, NEG, D, K, I_NUM):
    assert B % _NW == 0
    BPW = B // _NW
    NJG = (NEG + _L - 1) // _L
    NEG_PAD = NJG * _L

    mesh = plsc.VectorSubcoreMesh(
        core_axis_name="c", subcore_axis_name="s",
        num_cores=_NC, num_subcores=_NS)

    out_type = (
        jax.ShapeDtypeStruct((B * _L,), jnp.float32),
        jax.ShapeDtypeStruct((B * _L,), jnp.float32),
        jax.ShapeDtypeStruct((B * NEG,), jnp.float32),
        jax.ShapeDtypeStruct((B * NEG,), jnp.float32),
        jax.ShapeDtypeStruct((B,), jnp.float32),
        jax.ShapeDtypeStruct((B,), jnp.float32),
    )
    scratch = [
        pltpu.VMEM((BPW,), jnp.int32),
        pltpu.VMEM((BPW,), jnp.int32),
        pltpu.VMEM((BPW,), jnp.int32),
        pltpu.VMEM((BPW * D,), jnp.float32),
        pltpu.VMEM((K, BPW), jnp.int32),
        pltpu.VMEM((K, BPW), jnp.float32),
        pltpu.VMEM((BPW,), jnp.float32),
        pltpu.VMEM((BPW,), jnp.float32),
        pltpu.VMEM((BPW * NEG,), jnp.int32),
        pltpu.VMEM((NEG, _DP), jnp.float32),
        pltpu.VMEM((NEG, _DP), jnp.float32),
        pltpu.VMEM((NEG_PAD,), jnp.float32),
        pltpu.VMEM((NEG_PAD,), jnp.float32),
        pltpu.VMEM((NEG,), jnp.float32),
        pltpu.VMEM((NEG,), jnp.float32),
        pltpu.VMEM((_L,), jnp.int32),
        pltpu.VMEM((_L,), jnp.int32),
        pltpu.VMEM((_L, _DP), jnp.float32),
        pltpu.VMEM((_L, _DP), jnp.float32),
        pltpu.VMEM((BPW * _L,), jnp.float32),
        pltpu.VMEM((BPW * _L,), jnp.float32),
        pltpu.SemaphoreType.DMA,
        pltpu.SemaphoreType.DMA,
        pltpu.SemaphoreType.DMA,
        pltpu.SemaphoreType.DMA,
    ]

    def body(users_h, pos_h, neg_h, uep_h, iep_h, bu_h, bi_h, iicf_h, iinf_h,
             posinner_o, simpad_o, negsc_o, betain_o, betau_o, betaip_o,
             users_v, pos_v, idxtmp_v, urows_v, nbmat_v, simmat_v,
             betau_v, betaip_v, negidx_all,
             negrowsA, negrowsB, negscA, negscB, betainA, betainB,
             combidxA, combidxB, pirowsA, pirowsB,
             posinner_v, simpad_v, semA, semB, outsem, csem):
        wid = lax.axis_index("c") * _NS + lax.axis_index("s")
        base = wid * BPW
        lane = lax.iota(jnp.int32, _L)

        pltpu.sync_copy(users_h.at[pl.ds(base, BPW)], users_v)
        pltpu.sync_copy(pos_h.at[pl.ds(base, BPW)], pos_v)
        pltpu.sync_copy(neg_h.at[pl.ds(base * NEG, BPW * NEG)], negidx_all)
        c1 = pltpu.async_copy(uep_h.at[users_v], negrowsA.at[pl.ds(0, BPW)],
                              csem)
        c4 = pltpu.async_copy(bu_h.at[users_v], betau_v, csem)
        c5 = pltpu.async_copy(bi_h.at[pos_v], betaip_v, csem)
        c1.wait(); c4.wait(); c5.wait()

        def u_compact(i, carry):
            for t in range(D // _L):
                urows_v[pl.ds(i * D + t * _L, _L)] = (
                    negrowsA[i, pl.ds(t * _L, _L)])
            return carry

        lax.fori_loop(0, BPW, u_compact, 0)
        for k in range(K):
            for t in range(BPW // _L):
                sl = pl.ds(t * _L, _L)
                idxtmp_v[sl] = pos_v[sl] + jnp.full((_L,), k * I_NUM, jnp.int32)
            gn = pltpu.async_copy(iinf_h.at[idxtmp_v], nbmat_v.at[k], csem)
            gs = pltpu.async_copy(iicf_h.at[idxtmp_v], simmat_v.at[k], csem)
            gn.wait(); gs.wait()
        pltpu.sync_copy(betau_v, betau_o.at[pl.ds(base, BPW)])
        pltpu.sync_copy(betaip_v, betaip_o.at[pl.ds(base, BPW)])

        rowidx = [jnp.minimum(jg * _L + lane, NEG - 1) for jg in range(NJG)]
        nbcol = jnp.clip(lane - 1, 0, K - 1)
        nbmask = (lane >= 1) & (lane <= K)
        N2 = NEG - 128

        def descs(bl, negrows_x, betain_x, combidx_x, pirows_x, sem_x):
            i1 = negidx_all.at[pl.ds(bl * NEG, 128)]
            i2 = negidx_all.at[pl.ds(bl * NEG + 128, N2)]
            return (
                pltpu.make_async_copy(iep_h.at[i1],
                                      negrows_x.at[pl.ds(0, 128)], sem_x),
                pltpu.make_async_copy(iep_h.at[i2],
                                      negrows_x.at[pl.ds(128, N2)], sem_x),
                pltpu.make_async_copy(bi_h.at[i1],
                                      betain_x.at[pl.ds(0, 128)], sem_x),
                pltpu.make_async_copy(bi_h.at[i2],
                                      betain_x.at[pl.ds(128, N2)], sem_x),
                pltpu.make_async_copy(iep_h.at[combidx_x], pirows_x, sem_x),
            )

        def stage(bl, negrows_x, betain_x, combidx_x, pirows_x, sem_x):
            bl_vec = jnp.full((_L,), bl, jnp.int32)
            cand = plsc.load_gather(nbmat_v, [nbcol, bl_vec])
            posb = plsc.load_gather(pos_v, [bl_vec])
            combidx_x[...] = jnp.where(nbmask, cand, posb)
            for c in descs(bl, negrows_x, betain_x, combidx_x, pirows_x,
                           sem_x):
                c.start()

        def compute(bl, bl2, negrows_x, betain_x, combidx_x, pirows_x,
                    negsc_x, sem_x):
            bg = base + bl
            bl_vec = jnp.full((_L,), bl, jnp.int32)
            simv = plsc.load_gather(simmat_v, [nbcol, bl_vec])
            simpad_v[pl.ds(bl * _L, _L)] = jnp.where(nbmask, simv, 0.0)

            for c in descs(bl, negrows_x, betain_x, combidx_x, pirows_x,
                           sem_x):
                c.wait()

            accs = [jnp.zeros((_L,), jnp.float32) for _ in range(NJG)]
            pacc = jnp.zeros((_L,), jnp.float32)
            for t in range(D // _L):
                uvec = urows_v[pl.ds(bl * D + t * _L, _L)]
                for dd in range(_L):
                    d = t * _L + dd
                    u_d = uvec[dd]
                    dvec = jnp.full((_L,), d, jnp.int32)
                    for jg in range(NJG):
                        v = plsc.load_gather(negrows_x, [rowidx[jg], dvec])
                        accs[jg] = accs[jg] + v * u_d
                    pv = plsc.load_gather(pirows_x, [lane, dvec])
                    pacc = pacc + pv * u_d

            @pl.when(bl2 > 0)
            def _():
                pltpu.make_async_copy(
                    negsc_x.at[pl.ds(0, NEG)],
                    negsc_o.at[pl.ds(bg * NEG, NEG)], outsem).wait()
                pltpu.make_async_copy(
                    betain_x, betain_o.at[pl.ds(bg * NEG, NEG)], outsem).wait()

            for jg in range(NJG):
                negsc_x[pl.ds(jg * _L, _L)] = accs[jg]
            posinner_v[pl.ds(bl * _L, _L)] = pacc

            pltpu.async_copy(negsc_x.at[pl.ds(0, NEG)],
                             negsc_o.at[pl.ds(bg * NEG, NEG)], outsem)
            pltpu.async_copy(betain_x, betain_o.at[pl.ds(bg * NEG, NEG)],
                             outsem)

        stage(0, negrowsA, betainA, combidxA, pirowsA, semA)

        def b_pair(bl2, carry):
            bl = bl2 * 2
            stage(bl + 1, negrowsB, betainB, combidxB, pirowsB, semB)
            compute(bl, bl2, negrowsA, betainA, combidxA, pirowsA,
                    negscA, semA)

            @pl.when(bl + 2 < BPW)
            def _():
                stage(bl + 2, negrowsA, betainA, combidxA, pirowsA, semA)
            compute(bl + 1, bl2, negrowsB, betainB, combidxB, pirowsB,
                    negscB, semB)
            return carry

        lax.fori_loop(0, BPW // 2, b_pair, 0)

        blast = BPW - 2
        pltpu.make_async_copy(
            negscA.at[pl.ds(0, NEG)],
            negsc_o.at[pl.ds((base + blast) * NEG, NEG)], outsem).wait()
        pltpu.make_async_copy(
            betainA, betain_o.at[pl.ds((base + blast) * NEG, NEG)],
            outsem).wait()
        pltpu.make_async_copy(
            negscB.at[pl.ds(0, NEG)],
            negsc_o.at[pl.ds((base + blast + 1) * NEG, NEG)], outsem).wait()
        pltpu.make_async_copy(
            betainB, betain_o.at[pl.ds((base + blast + 1) * NEG, NEG)],
            outsem).wait()

        pltpu.sync_copy(posinner_v, posinner_o.at[pl.ds(base * _L, BPW * _L)])
        pltpu.sync_copy(simpad_v, simpad_o.at[pl.ds(base * _L, BPW * _L)])

    return pl.kernel(body, out_type=out_type, mesh=mesh,
                     scratch_types=scratch,
                     compiler_params=pltpu.CompilerParams(
                         needs_layout_passes=False))


def _make_norm_body(ncols, block):
    def _norm_body(ue_r, ie_r, out_r):
        g = pl.program_id(0)
        ng = pl.num_programs(0)

        @pl.when(g == 0)
        def _init():
            out_r[0, 0] = 0.0

        x = ue_r[...]
        y = ie_r[...]

        @pl.when(g < ng - 1)
        def _full():
            out_r[0, 0] += jnp.sum(x * x) + jnp.sum(y * y)

        @pl.when(g == ng - 1)
        def _masked():
            col = g * block + lax.broadcasted_iota(jnp.int32, x.shape, 1)
            valid = col < ncols
            out_r[0, 0] += (jnp.sum(jnp.where(valid, x * x, 0.0))
                            + jnp.sum(jnp.where(valid, y * y, 0.0)))
    return _norm_body


def _final_body(betau_r, betaip_r, posinner_r, simpad_r, negsc_r, betain_r,
                norm_r, out_r):
    bu = betau_r[...]
    bip = betaip_r[...]
    pin = posinner_r[...]
    simp = simpad_r[...]
    ns = negsc_r[...]
    bin_ = betain_r[...]
    NEG = ns.shape[1]

    neg_w = W3 + W4 * bu * bin_
    neg_term = (NEG_WEIGHT / NEG) * jnp.sum(neg_w * _softplus(ns))

    pow_w = W1 + W2 * bu * bip
    col = lax.broadcasted_iota(jnp.int32, pin.shape, 1)
    wmat = jnp.where(col == 0, pow_w, 0.0) + LAMBDA * simp
    pos_term = jnp.sum(wmat * _softplus(-pin))

    out_r[0, 0] = pos_term + neg_term + GAMMA * 0.5 * norm_r[0, 0]


def kernel(users, pos_items, neg_items, user_embeds, item_embeds,
           beta_uD, beta_iD, ii_constraint_mat, ii_neighbor_mat):
    B = users.shape[0]
    NEG = neg_items.shape[1]
    D = user_embeds.shape[1]
    K = ii_neighbor_mat.shape[1]
    U_NUM = user_embeds.shape[0]
    I_NUM = item_embeds.shape[0]

    users = users.astype(jnp.int32)
    pos_items = pos_items.astype(jnp.int32)
    neg_flat = neg_items.reshape(-1).astype(jnp.int32)

    ue_pad = jnp.pad(user_embeds, ((0, 0), (0, _DP - D)))
    ie_pad = jnp.pad(item_embeds, ((0, 0), (0, _DP - D)))
    iin_flat = ii_neighbor_mat.astype(jnp.int32).T.reshape(-1)
    iic_flat = ii_constraint_mat.T.reshape(-1)

    sc = _make_sc_kernel(B, NEG, D, K, I_NUM)
    posinner, simpad, negsc, betain, betau, betaip = sc(
        users, pos_items, neg_flat, ue_pad, ie_pad,
        beta_uD, beta_iD, iic_flat, iin_flat)

    assert U_NUM == I_NUM
    BLK = 128 * 61
    G = -(-U_NUM // BLK)
    norm = pl.pallas_call(
        _make_norm_body(U_NUM, BLK),
        grid=(G,),
        in_specs=[
            pl.BlockSpec((D, BLK), lambda g: (0, g)),
            pl.BlockSpec((D, BLK), lambda g: (0, g)),
        ],
        out_specs=pl.BlockSpec((1, 1), lambda g: (0, 0),
                               memory_space=pltpu.SMEM),
        out_shape=jax.ShapeDtypeStruct((1, 1), jnp.float32),
    )(user_embeds.T, item_embeds.T)

    full = lambda s: pl.BlockSpec(s, lambda: tuple(0 for _ in s))
    loss = pl.pallas_call(
        _final_body,
        in_specs=[
            full((B, 1)), full((B, 1)), full((B, _L)), full((B, _L)),
            full((B, NEG)), full((B, NEG)),
            pl.BlockSpec((1, 1), lambda: (0, 0), memory_space=pltpu.SMEM),
        ],
        out_specs=pl.BlockSpec((1, 1), lambda: (0, 0),
                               memory_space=pltpu.SMEM),
        out_shape=jax.ShapeDtypeStruct((1, 1), jnp.float32),
    )(betau.reshape(B, 1), betaip.reshape(B, 1), posinner.reshape(B, _L),
      simpad.reshape(B, _L), negsc.reshape(B, NEG), betain.reshape(B, NEG),
      norm)

    return loss[0, 0]
